# Initial kernel scaffold; baseline (speedup 1.0000x reference)
import jax
import jax.numpy as jnp
from jax.experimental import pallas as pl


def kernel(x, padded_node_mask, indices_subnodes, node_num, padded_edge_mask, time_entirenodes_emdim, ln1_g, ln1_b, lnf_g, lnf_b, W1, b1, W2, b2):
    raise NotImplementedError("write your pallas kernel here")


# speedup vs baseline: 1.5392x; 1.5392x over previous
import jax
import jax.numpy as jnp
from jax.experimental import pallas as pl
from jax.experimental.pallas import tpu as pltpu

T, NT, D = 4, 16384, 128
N, CL, CD = 50000, 8, 64
CH = N // CL


def _ln_body(x_ref, g_ref, b_ref, o_ref):
    x = x_ref[...]
    m = jnp.mean(x, axis=-1, keepdims=True)
    v = jnp.mean((x - m) ** 2, axis=-1, keepdims=True)
    o_ref[...] = (x - m) * jax.lax.rsqrt(v + 1e-5) * g_ref[...] + b_ref[...]


def _ln_rows(x2d, g, b, block_rows):
    rows = x2d.shape[0]
    grid = rows // block_rows
    return pl.pallas_call(
        _ln_body,
        grid=(grid,),
        in_specs=[
            pl.BlockSpec((block_rows, x2d.shape[1]), lambda i: (i, 0)),
            pl.BlockSpec((1, x2d.shape[1]), lambda i: (0, 0)),
            pl.BlockSpec((1, x2d.shape[1]), lambda i: (0, 0)),
        ],
        out_specs=pl.BlockSpec((block_rows, x2d.shape[1]), lambda i: (i, 0)),
        out_shape=jax.ShapeDtypeStruct(x2d.shape, jnp.float32),
    )(x2d, g.reshape(1, -1), b.reshape(1, -1))


def _mlp_body(e_ref, w1_ref, b1_ref, w2_ref, b2_ref, o_ref):
    x = e_ref[0, 0]
    h = jnp.dot(x, w1_ref[...], preferred_element_type=jnp.float32) + b1_ref[...]
    h = 0.5 * h * (1.0 + jax.lax.erf(h * 0.7071067811865476))
    s = jnp.sum(h, axis=0, keepdims=True)
    o_ref[...] = (
        jnp.dot(s, w2_ref[...], preferred_element_type=jnp.float32)
        + CH * b2_ref[...]
    )[None, None]


def _mlp_reduce(entire, W1, b1, W2, b2):
    e4 = entire.reshape(T, CL, CH, D)
    out = pl.pallas_call(
        _mlp_body,
        grid=(T, CL),
        in_specs=[
            pl.BlockSpec((1, 1, CH, D), lambda t, c: (t, c, 0, 0)),
            pl.BlockSpec((D, 2 * CD), lambda t, c: (0, 0)),
            pl.BlockSpec((1, 2 * CD), lambda t, c: (0, 0)),
            pl.BlockSpec((2 * CD, CD), lambda t, c: (0, 0)),
            pl.BlockSpec((1, CD), lambda t, c: (0, 0)),
        ],
        out_specs=pl.BlockSpec((1, 1, 1, CD), lambda t, c: (t, c, 0, 0)),
        out_shape=jax.ShapeDtypeStruct((T, CL, 1, CD), jnp.float32),
    )(e4, W1, b1.reshape(1, -1), W2, b2.reshape(1, -1))
    return out.reshape(T, CL * CD)


def kernel(x, padded_node_mask, indices_subnodes, node_num, padded_edge_mask,
           time_entirenodes_emdim, ln1_g, ln1_b, lnf_g, lnf_b, W1, b1, W2, b2):
    xf = x.reshape(T * NT, D)
    y = _ln_rows(xf, ln1_g, ln1_b, 2048)

    t_of_tok = jnp.arange(T * NT, dtype=jnp.int32) // NT
    flat_idx = t_of_tok * N + indices_subnodes.astype(jnp.int32)
    entire = time_entirenodes_emdim.reshape(T * N, D).at[flat_idx].add(y)

    compressed = _mlp_reduce(entire, W1, b1, W2, b2)
    out = _ln_rows(compressed, lnf_g, lnf_b, T)
    return out.reshape(T, 1, CL * CD)
